# baseline (speedup 1.0000x reference)
import functools

import jax
import jax.numpy as jnp
from jax import lax
from jax.experimental import pallas as pl
from jax.experimental.pallas import tpu as pltpu
from jax.experimental.pallas import tpu_sc as plsc

T, B, D = 2048, 4, 1024
NC, NS, L = 2, 16, 16
CHUNK = 8
VECS = D // L
CH_A, CH_B = 10, 6
R_A, R_B = CH_A * CHUNK, CH_B * CHUNK
SPLIT = NS * R_A


def _sc_body(x_hbm, pos_hbm, out_hbm, x_v, pos_v, in_sems, out_sems):
    cid = lax.axis_index("c")
    sid = lax.axis_index("s")

    def start_in(t0, slot):
        pltpu.async_copy(x_hbm.at[pl.ds(t0, CHUNK)], x_v.at[slot], in_sems.at[slot])
        pltpu.async_copy(pos_hbm.at[pl.ds(t0, CHUNK)], pos_v.at[slot], in_sems.at[slot])

    def wait_in(slot):
        pltpu.make_async_copy(x_hbm.at[pl.ds(0, CHUNK)], x_v.at[slot], in_sems.at[slot]).wait()
        pltpu.make_async_copy(pos_hbm.at[pl.ds(0, CHUNK)], pos_v.at[slot], in_sems.at[slot]).wait()

    def start_out(t0, slot):
        pltpu.async_copy(x_v.at[slot], out_hbm.at[pl.ds(t0, CHUNK)], out_sems.at[slot])

    def wait_out(slot):
        pltpu.make_async_copy(x_v.at[slot], out_hbm.at[pl.ds(0, CHUNK)], out_sems.at[slot]).wait()

    def compute(slot):
        def row_body(t, carry):
            for j in range(VECS):
                p = pos_v[slot, t, pl.ds(j * L, L)]
                for b in range(B):
                    plsc.addupdate(x_v.at[slot, t, b, pl.ds(j * L, L)], p)
            return carry

        lax.fori_loop(0, CHUNK, row_body, 0)

    def pipeline(base, nchunk):
        start_in(base, 0)
        for c in range(nchunk):
            slot = c % 2
            if c + 1 < nchunk:
                if c >= 1:
                    wait_out(1 - slot)
                start_in(base + (c + 1) * CHUNK, 1 - slot)
            wait_in(slot)
            compute(slot)
            start_out(base + c * CHUNK, slot)
        wait_out(0)
        wait_out(1)

    @pl.when(cid == 0)
    def _():
        pipeline(sid * R_A, CH_A)

    @pl.when(cid != 0)
    def _():
        pipeline(SPLIT + sid * R_B, CH_B)


def kernel(x, pos_table):
    mesh = plsc.VectorSubcoreMesh(core_axis_name="c", subcore_axis_name="s")
    k = functools.partial(
        pl.kernel,
        mesh=mesh,
        out_type=jax.ShapeDtypeStruct((T, B, D), jnp.float32),
        scratch_types=[
            pltpu.VMEM((2, CHUNK, B, D), jnp.float32),
            pltpu.VMEM((2, CHUNK, D), jnp.float32),
            pltpu.SemaphoreType.DMA((2,)),
            pltpu.SemaphoreType.DMA((2,)),
        ],
    )(_sc_body)
    return k(x, pos_table)


# speedup vs baseline: 1.0108x; 1.0108x over previous
import functools

import jax
import jax.numpy as jnp
from jax import lax
from jax.experimental import pallas as pl
from jax.experimental.pallas import tpu as pltpu
from jax.experimental.pallas import tpu_sc as plsc

T, B, D = 2048, 4, 1024
NC, NS, L = 2, 16, 16
CHUNK = 8
VECS = D // L
CH_A, CH_B = 10, 6
R_A, R_B = CH_A * CHUNK, CH_B * CHUNK
SPLIT = NS * R_A


def _sc_body(x_hbm, pos_hbm, out_hbm, x_v, pos_v, in_sems, out_sems):
    cid = lax.axis_index("c")
    sid = lax.axis_index("s")

    def start_in(t0, slot):
        pltpu.async_copy(x_hbm.at[pl.ds(t0, CHUNK)], x_v.at[slot], in_sems.at[slot])
        pltpu.async_copy(pos_hbm.at[pl.ds(t0, CHUNK)], pos_v.at[slot], in_sems.at[slot])

    def wait_in(slot):
        pltpu.make_async_copy(x_hbm.at[pl.ds(0, CHUNK)], x_v.at[slot], in_sems.at[slot]).wait()
        pltpu.make_async_copy(pos_hbm.at[pl.ds(0, CHUNK)], pos_v.at[slot], in_sems.at[slot]).wait()

    def start_out(t0, slot):
        pltpu.async_copy(x_v.at[slot], out_hbm.at[pl.ds(t0, CHUNK)], out_sems.at[slot])

    def wait_out(slot):
        pltpu.make_async_copy(x_v.at[slot], out_hbm.at[pl.ds(0, CHUNK)], out_sems.at[slot]).wait()

    def compute(slot):
        def row_body(t, carry):
            for j in range(VECS):
                p = pos_v[slot, t, pl.ds(j * L, L)]
                for b in range(B):
                    plsc.addupdate(x_v.at[slot, t, b, pl.ds(j * L, L)], p)
            return carry

        lax.fori_loop(0, CHUNK, row_body, 0)

    def pipeline(base, nchunk):
        start_in(base, 0)
        for c in range(nchunk):
            slot = c % 2
            if c + 1 < nchunk:
                if c >= 1:
                    wait_out(1 - slot)
                start_in(base + (c + 1) * CHUNK, 1 - slot)
            wait_in(slot)
            compute(slot)
            start_out(base + c * CHUNK, slot)
        wait_out(0)
        wait_out(1)

    @pl.when(cid != 0)
    def _():
        pipeline(sid * R_A, CH_A)

    @pl.when(cid == 0)
    def _():
        pipeline(SPLIT + sid * R_B, CH_B)


def kernel(x, pos_table):
    mesh = plsc.VectorSubcoreMesh(core_axis_name="c", subcore_axis_name="s")
    k = functools.partial(
        pl.kernel,
        mesh=mesh,
        out_type=jax.ShapeDtypeStruct((T, B, D), jnp.float32),
        scratch_types=[
            pltpu.VMEM((2, CHUNK, B, D), jnp.float32),
            pltpu.VMEM((2, CHUNK, D), jnp.float32),
            pltpu.SemaphoreType.DMA((2,)),
            pltpu.SemaphoreType.DMA((2,)),
        ],
    )(_sc_body)
    return k(x, pos_table)


# speedup vs baseline: 1.2096x; 1.1967x over previous
import functools

import jax
import jax.numpy as jnp
from jax import lax
from jax.experimental import pallas as pl
from jax.experimental.pallas import tpu as pltpu
from jax.experimental.pallas import tpu_sc as plsc

T, B, D = 2048, 4, 1024
NC, NS, L = 2, 16, 16
CHUNK = 8
VECS = D // L
CH_SYM = T // (NC * NS) // CHUNK


def _sc_body(x_hbm, pos_hbm, out_hbm, x_v, pos_v, in_sems, out_sems):
    cid = lax.axis_index("c")
    sid = lax.axis_index("s")

    def start_in(t0, slot):
        pltpu.async_copy(x_hbm.at[pl.ds(t0, CHUNK)], x_v.at[slot], in_sems.at[slot])
        pltpu.async_copy(pos_hbm.at[pl.ds(t0, CHUNK)], pos_v.at[slot], in_sems.at[slot])

    def wait_in(slot):
        pltpu.make_async_copy(x_hbm.at[pl.ds(0, CHUNK)], x_v.at[slot], in_sems.at[slot]).wait()
        pltpu.make_async_copy(pos_hbm.at[pl.ds(0, CHUNK)], pos_v.at[slot], in_sems.at[slot]).wait()

    def start_out(t0, slot):
        pltpu.async_copy(x_v.at[slot], out_hbm.at[pl.ds(t0, CHUNK)], out_sems.at[slot])

    def wait_out(slot):
        pltpu.make_async_copy(x_v.at[slot], out_hbm.at[pl.ds(0, CHUNK)], out_sems.at[slot]).wait()

    def compute(slot):
        def row_body(t, carry):
            for j in range(VECS):
                p = pos_v[slot, t, pl.ds(j * L, L)]
                for b in range(B):
                    plsc.addupdate(x_v.at[slot, t, b, pl.ds(j * L, L)], p)
            return carry

        lax.fori_loop(0, CHUNK, row_body, 0)

    def pipeline(base, nchunk):
        start_in(base, 0)
        for c in range(nchunk):
            slot = c % 2
            if c + 1 < nchunk:
                if c >= 1:
                    wait_out(1 - slot)
                start_in(base + (c + 1) * CHUNK, 1 - slot)
            wait_in(slot)
            compute(slot)
            start_out(base + c * CHUNK, slot)
        wait_out(0)
        wait_out(1)

    wid = sid * NC + cid
    pipeline(wid * (CH_SYM * CHUNK), CH_SYM)


def kernel(x, pos_table):
    mesh = plsc.VectorSubcoreMesh(core_axis_name="c", subcore_axis_name="s")
    k = functools.partial(
        pl.kernel,
        mesh=mesh,
        out_type=jax.ShapeDtypeStruct((T, B, D), jnp.float32),
        scratch_types=[
            pltpu.VMEM((2, CHUNK, B, D), jnp.float32),
            pltpu.VMEM((2, CHUNK, D), jnp.float32),
            pltpu.SemaphoreType.DMA((2,)),
            pltpu.SemaphoreType.DMA((2,)),
        ],
    )(_sc_body)
    return k(x, pos_table)
